# Initial kernel scaffold; baseline (speedup 1.0000x reference)
import jax
import jax.numpy as jnp
from jax.experimental import pallas as pl


def kernel(embedding, token_table, pos_table, gamma, beta):
    raise NotImplementedError("write your pallas kernel here")


# speedup vs baseline: 3.3767x; 3.3767x over previous
import functools

import jax
import jax.numpy as jnp
from jax import lax
from jax.experimental import pallas as pl
from jax.experimental.pallas import tpu as pltpu
from jax.experimental.pallas import tpu_sc as plsc

VOCAB = 100000
DIM = 128
MAX_LEN = 200
B = 1024
L = 200

NC = 2
NS = 16
NW = NC * NS
N_ROWS = B * L
ROWS_PER_W = N_ROWS // NW
CHUNK = 128
CHUNKS_PER_W = ROWS_PER_W // CHUNK
NVR = DIM // 16
GRP = 16
NGRP = CHUNK // GRP

_DNUMS = lax.GatherDimensionNumbers(
    offset_dims=(), collapsed_slice_dims=(0,), start_index_map=(0,))


def _gather16(v, idx):
    return lax.gather(v, idx[:, None], _DNUMS, (1,),
                      mode=lax.GatherScatterMode.PROMISE_IN_BOUNDS)


def _lane_sum(v, lane):
    for sh in (8, 4, 2, 1):
        v = v + _gather16(v, lane ^ sh)
    return v


def _vrsqrt(y):
    r = jnp.ones((16,), jnp.float32)
    for p in (32, 16, 8, 4, 2, 1):
        big = y >= jnp.float32(4.0 ** p)
        y = jnp.where(big, y * jnp.float32(4.0 ** (-p)), y)
        r = jnp.where(big, r * jnp.float32(2.0 ** (-p)), r)
    for p in (32, 16, 8, 4, 2, 1):
        small = y < jnp.float32(4.0 ** (1 - p))
        y = jnp.where(small, y * jnp.float32(4.0 ** p), y)
        r = jnp.where(small, r * jnp.float32(2.0 ** p), r)
    r0 = jnp.float32(7.0 / 6.0) - y * jnp.float32(1.0 / 6.0)
    half = y * jnp.float32(0.5)
    for _ in range(4):
        r0 = r0 * (jnp.float32(1.5) - half * r0 * r0)
    return r0 * r


def _body(emb_hbm, tok_hbm, pos_hbm, gam_hbm, bet_hbm, out_hbm,
          idx_v, rows_v, pos_v, gam_v, bet_v, sem):
    cid = lax.axis_index("c")
    sid = lax.axis_index("s")
    wid = sid * NC + cid
    base_chunk = wid * CHUNKS_PER_W

    pltpu.sync_copy(emb_hbm.at[wid], idx_v)
    pltpu.sync_copy(pos_hbm, pos_v)
    pltpu.sync_copy(gam_hbm, gam_v)
    pltpu.sync_copy(bet_hbm, bet_v)

    gb = [gam_v[pl.ds(k * 16, 16)] for k in range(NVR)]
    bb = [bet_v[pl.ds(k * 16, 16)] for k in range(NVR)]

    inv_d = jnp.float32(1.0 / DIM)
    lane = lax.iota(jnp.int32, 16)
    zf = jnp.zeros((16,), jnp.float32)

    def chunk_body(c, carry):
        pltpu.async_copy(tok_hbm.at[idx_v.at[c]], rows_v, sem).wait()
        l0 = lax.rem(c * CHUNK, MAX_LEN)

        def group_body(g, gcarry):
            base_j = g * GRP

            def p1(r, acc):
                meanacc, varacc = acc
                j = base_j + r
                l = l0 + j
                l = jnp.where(l >= MAX_LEN, l - MAX_LEN, l)
                x = [rows_v[j, pl.ds(k * 16, 16)] + pos_v[l, pl.ds(k * 16, 16)]
                     for k in range(NVR)]
                s = x[0]
                for k in range(1, NVR):
                    s = s + x[k]
                q = x[0] * x[0]
                for k in range(1, NVR):
                    q = q + x[k] * x[k]
                mean = _lane_sum(s, lane) * inv_d
                ex2 = _lane_sum(q, lane) * inv_d
                var = ex2 - mean * mean
                for k in range(NVR):
                    rows_v[j, pl.ds(k * 16, 16)] = x[k]
                rmask = lane == r
                return (jnp.where(rmask, mean, meanacc),
                        jnp.where(rmask, var, varacc))

            meanv, varv = lax.fori_loop(0, GRP, p1, (zf, zf))
            scalev = _vrsqrt(varv + jnp.float32(1e-12))

            def p2(r, rcarry):
                j = base_j + r
                idxsplat = (lane & 0) + r
                mean = _gather16(meanv, idxsplat)
                scale = _gather16(scalev, idxsplat)
                for k in range(NVR):
                    d = rows_v[j, pl.ds(k * 16, 16)] - mean
                    rows_v[j, pl.ds(k * 16, 16)] = d * (scale * gb[k]) + bb[k]
                return rcarry

            lax.fori_loop(0, GRP, p2, 0)
            return gcarry

        lax.fori_loop(0, NGRP, group_body, 0)
        row0 = (base_chunk + c) * CHUNK
        pltpu.sync_copy(rows_v, out_hbm.at[pl.ds(row0, CHUNK)])
        return carry

    lax.fori_loop(0, CHUNKS_PER_W, chunk_body, 0)


_sc_call = functools.partial(
    pl.kernel,
    mesh=plsc.VectorSubcoreMesh(core_axis_name="c", subcore_axis_name="s"),
    out_type=jax.ShapeDtypeStruct((N_ROWS, DIM), jnp.float32),
    scratch_types=[
        pltpu.VMEM((CHUNKS_PER_W, CHUNK), jnp.int32),
        pltpu.VMEM((CHUNK, DIM), jnp.float32),
        pltpu.VMEM((MAX_LEN, DIM), jnp.float32),
        pltpu.VMEM((DIM,), jnp.float32),
        pltpu.VMEM((DIM,), jnp.float32),
        pltpu.SemaphoreType.DMA,
    ],
)(_body)


def kernel(embedding, token_table, pos_table, gamma, beta):
    emb2 = embedding.astype(jnp.int32).reshape(NW, CHUNKS_PER_W, CHUNK)
    out = _sc_call(emb2, token_table, pos_table, gamma, beta)
    return out.reshape(B, L, DIM)


# speedup vs baseline: 3.9627x; 1.1735x over previous
import functools

import jax
import jax.numpy as jnp
from jax import lax
from jax.experimental import pallas as pl
from jax.experimental.pallas import tpu as pltpu
from jax.experimental.pallas import tpu_sc as plsc

VOCAB = 100000
DIM = 128
MAX_LEN = 200
B = 1024
L = 200

NC = 2
NS = 16
NW = NC * NS
N_ROWS = B * L
ROWS_PER_W = N_ROWS // NW
CHUNK = 128
CHUNKS_PER_W = ROWS_PER_W // CHUNK
NBUF = 5
ROUNDS = CHUNKS_PER_W // NBUF


def _gather_body(emb_hbm, tok_hbm, out_hbm, idx_v, rows_v, gsem, wsem):
    cid = lax.axis_index("c")
    sid = lax.axis_index("s")
    wid = sid * NC + cid
    base_chunk = wid * CHUNKS_PER_W

    pltpu.sync_copy(emb_hbm.at[wid], idx_v)

    def start_gather(b, c):
        pltpu.async_copy(tok_hbm.at[idx_v.at[c]], rows_v.at[b], gsem.at[b])

    def wait_gather(b, c):
        pltpu.make_async_copy(
            tok_hbm.at[idx_v.at[c]], rows_v.at[b], gsem.at[b]).wait()

    def start_write(b, c):
        row0 = (base_chunk + c) * CHUNK
        pltpu.async_copy(
            rows_v.at[b], out_hbm.at[pl.ds(row0, CHUNK)], wsem.at[b])

    def wait_write(b, c):
        row0 = (base_chunk + c) * CHUNK
        pltpu.make_async_copy(
            rows_v.at[b], out_hbm.at[pl.ds(row0, CHUNK)], wsem.at[b]).wait()

    for b in range(NBUF):
        start_gather(b, b)

    def round_body(r, carry):
        c0 = r * NBUF
        for b in range(NBUF):
            wait_gather(b, c0 + b)
            start_write(b, c0 + b)
        for b in range(NBUF):
            wait_write(b, c0 + b)
            start_gather(b, c0 + NBUF + b)
        return carry

    lax.fori_loop(0, ROUNDS - 1, round_body, 0)

    c0 = (ROUNDS - 1) * NBUF
    for b in range(NBUF):
        wait_gather(b, c0 + b)
        start_write(b, c0 + b)
    for b in range(NBUF):
        wait_write(b, c0 + b)


_gather_call = functools.partial(
    pl.kernel,
    mesh=plsc.VectorSubcoreMesh(core_axis_name="c", subcore_axis_name="s"),
    out_type=jax.ShapeDtypeStruct((N_ROWS, DIM), jnp.float32),
    scratch_types=[
        pltpu.VMEM((CHUNKS_PER_W, CHUNK), jnp.int32),
        pltpu.VMEM((NBUF, CHUNK, DIM), jnp.float32),
        pltpu.SemaphoreType.DMA((NBUF,)),
        pltpu.SemaphoreType.DMA((NBUF,)),
    ],
)(_gather_body)


ROWS_BLK = 1600
REP = ROWS_BLK // MAX_LEN


def _ln_body(g_ref, pos_ref, gam_ref, bet_ref, o_ref):
    x = g_ref[...] + jnp.tile(pos_ref[...], (REP, 1))
    mean = jnp.mean(x, axis=1, keepdims=True)
    d = x - mean
    var = jnp.mean(d * d, axis=1, keepdims=True)
    y = d / jnp.sqrt(var + jnp.float32(1e-12))
    o_ref[...] = y * gam_ref[...] + bet_ref[...]


_ln_call = pl.pallas_call(
    _ln_body,
    grid=(N_ROWS // ROWS_BLK,),
    in_specs=[
        pl.BlockSpec((ROWS_BLK, DIM), lambda i: (i, 0)),
        pl.BlockSpec((MAX_LEN, DIM), lambda i: (0, 0)),
        pl.BlockSpec((1, DIM), lambda i: (0, 0)),
        pl.BlockSpec((1, DIM), lambda i: (0, 0)),
    ],
    out_specs=pl.BlockSpec((ROWS_BLK, DIM), lambda i: (i, 0)),
    out_shape=jax.ShapeDtypeStruct((N_ROWS, DIM), jnp.float32),
)


def kernel(embedding, token_table, pos_table, gamma, beta):
    emb2 = embedding.astype(jnp.int32).reshape(NW, CHUNKS_PER_W, CHUNK)
    gathered = _gather_call(emb2, token_table)
    out = _ln_call(gathered, pos_table,
                   gamma.reshape(1, DIM), beta.reshape(1, DIM))
    return out.reshape(B, L, DIM)


# speedup vs baseline: 4.5607x; 1.1509x over previous
import functools

import jax
import jax.numpy as jnp
from jax import lax
from jax.experimental import pallas as pl
from jax.experimental.pallas import tpu as pltpu
from jax.experimental.pallas import tpu_sc as plsc

VOCAB = 100000
DIM = 128
MAX_LEN = 200
B = 1024
L = 200

NC = 2
NS = 16
NW = NC * NS
N_ROWS = B * L
ROWS_PER_W = N_ROWS // NW
CHUNK = 128
CHUNKS_PER_W = ROWS_PER_W // CHUNK
NBUF = 5
ROUNDS = CHUNKS_PER_W // NBUF


def _gather_body(emb_hbm, tok_hbm, out_hbm, idx_v, rows_v, gsem, wsem):
    cid = lax.axis_index("c")
    sid = lax.axis_index("s")
    wid = sid * NC + cid
    base_chunk = wid * CHUNKS_PER_W

    pltpu.sync_copy(emb_hbm.at[wid], idx_v)

    def start_gather(b, c):
        pltpu.async_copy(tok_hbm.at[idx_v.at[c]], rows_v.at[b], gsem.at[b])

    def wait_gather(b, c):
        pltpu.make_async_copy(
            tok_hbm.at[idx_v.at[c]], rows_v.at[b], gsem.at[b]).wait()

    def start_write(b, c):
        row0 = (base_chunk + c) * CHUNK
        pltpu.async_copy(
            rows_v.at[b], out_hbm.at[pl.ds(row0, CHUNK)], wsem.at[b])

    def wait_write(b, c):
        row0 = (base_chunk + c) * CHUNK
        pltpu.make_async_copy(
            rows_v.at[b], out_hbm.at[pl.ds(row0, CHUNK)], wsem.at[b]).wait()

    for b in range(NBUF):
        start_gather(b, b)

    def round_body(r, carry):
        c0 = r * NBUF
        for b in range(NBUF):
            wait_gather(b, c0 + b)
            start_write(b, c0 + b)
        for b in range(NBUF):
            wait_write(b, c0 + b)
            start_gather(b, c0 + NBUF + b)
        return carry

    lax.fori_loop(0, ROUNDS - 1, round_body, 0)

    c0 = (ROUNDS - 1) * NBUF
    for b in range(NBUF):
        wait_gather(b, c0 + b)
        start_write(b, c0 + b)
    for b in range(NBUF):
        wait_write(b, c0 + b)


_gather_call = functools.partial(
    pl.kernel,
    mesh=plsc.VectorSubcoreMesh(core_axis_name="c", subcore_axis_name="s"),
    out_type=jax.ShapeDtypeStruct((N_ROWS, DIM), jnp.float32),
    scratch_types=[
        pltpu.VMEM((CHUNKS_PER_W, CHUNK), jnp.int32),
        pltpu.VMEM((NBUF, CHUNK, DIM), jnp.float32),
        pltpu.SemaphoreType.DMA((NBUF,)),
        pltpu.SemaphoreType.DMA((NBUF,)),
    ],
)(_gather_body)


ROWS_BLK = 3200
REP = ROWS_BLK // MAX_LEN


def _ln_body(g_ref, pos_ref, gam_ref, bet_ref, o_ref):
    x = g_ref[...] + jnp.tile(pos_ref[...], (REP, 1))
    mean = jnp.mean(x, axis=1, keepdims=True)
    d = x - mean
    var = jnp.mean(d * d, axis=1, keepdims=True)
    y = d / jnp.sqrt(var + jnp.float32(1e-12))
    o_ref[...] = y * gam_ref[...] + bet_ref[...]


_ln_call = pl.pallas_call(
    _ln_body,
    grid=(N_ROWS // ROWS_BLK,),
    in_specs=[
        pl.BlockSpec((ROWS_BLK, DIM), lambda i: (i, 0)),
        pl.BlockSpec((MAX_LEN, DIM), lambda i: (0, 0)),
        pl.BlockSpec((1, DIM), lambda i: (0, 0)),
        pl.BlockSpec((1, DIM), lambda i: (0, 0)),
    ],
    out_specs=pl.BlockSpec((ROWS_BLK, DIM), lambda i: (i, 0)),
    out_shape=jax.ShapeDtypeStruct((N_ROWS, DIM), jnp.float32),
)


def kernel(embedding, token_table, pos_table, gamma, beta):
    emb2 = embedding.astype(jnp.int32).reshape(NW, CHUNKS_PER_W, CHUNK)
    gathered = _gather_call(emb2, token_table)
    out = _ln_call(gathered, pos_table,
                   gamma.reshape(1, DIM), beta.reshape(1, DIM))
    return out.reshape(B, L, DIM)


# speedup vs baseline: 4.9553x; 1.0865x over previous
import functools

import jax
import jax.numpy as jnp
from jax import lax
from jax.experimental import pallas as pl
from jax.experimental.pallas import tpu as pltpu
from jax.experimental.pallas import tpu_sc as plsc

VOCAB = 100000
DIM = 128
MAX_LEN = 200
B = 1024
L = 200

NC = 2
NS = 16
NW = NC * NS
N_ROWS = B * L
ROWS_PER_W = N_ROWS // NW
CHUNK = 128
CHUNKS_PER_W = ROWS_PER_W // CHUNK
NBUF = 5
ROUNDS = CHUNKS_PER_W // NBUF


def _gather_body(emb_hbm, tok_hbm, out_hbm, idx_v, rows_v, gsem, wsem):
    cid = lax.axis_index("c")
    sid = lax.axis_index("s")
    wid = sid * NC + cid
    base_chunk = wid * CHUNKS_PER_W

    pltpu.sync_copy(emb_hbm.at[wid], idx_v)

    def start_gather(b, c):
        pltpu.async_copy(tok_hbm.at[idx_v.at[c]], rows_v.at[b], gsem.at[b])

    def wait_gather(b, c):
        pltpu.make_async_copy(
            tok_hbm.at[idx_v.at[c]], rows_v.at[b], gsem.at[b]).wait()

    def start_write(b, c):
        row0 = (base_chunk + c) * CHUNK
        pltpu.async_copy(
            rows_v.at[b], out_hbm.at[pl.ds(row0, CHUNK)], wsem.at[b])

    def wait_write(b, c):
        row0 = (base_chunk + c) * CHUNK
        pltpu.make_async_copy(
            rows_v.at[b], out_hbm.at[pl.ds(row0, CHUNK)], wsem.at[b]).wait()

    for b in range(NBUF):
        start_gather(b, b)

    def round_body(r, carry):
        c0 = r * NBUF
        for b in range(NBUF):
            wait_gather(b, c0 + b)
            start_write(b, c0 + b)
        for b in range(NBUF):
            wait_write(b, c0 + b)
            start_gather(b, c0 + NBUF + b)
        return carry

    lax.fori_loop(0, ROUNDS - 1, round_body, 0)

    c0 = (ROUNDS - 1) * NBUF
    for b in range(NBUF):
        wait_gather(b, c0 + b)
        start_write(b, c0 + b)
    for b in range(NBUF):
        wait_write(b, c0 + b)


_gather_call = functools.partial(
    pl.kernel,
    mesh=plsc.VectorSubcoreMesh(core_axis_name="c", subcore_axis_name="s"),
    out_type=jax.ShapeDtypeStruct((N_ROWS, DIM), jnp.float32),
    scratch_types=[
        pltpu.VMEM((CHUNKS_PER_W, CHUNK), jnp.int32),
        pltpu.VMEM((NBUF, CHUNK, DIM), jnp.float32),
        pltpu.SemaphoreType.DMA((NBUF,)),
        pltpu.SemaphoreType.DMA((NBUF,)),
    ],
)(_gather_body)


ROWS_BLK = 6400
REP = ROWS_BLK // MAX_LEN


def _ln_body(g_ref, pos_ref, gam_ref, bet_ref, o_ref):
    x = g_ref[...] + jnp.tile(pos_ref[...], (REP, 1))
    mean = jnp.mean(x, axis=1, keepdims=True)
    d = x - mean
    var = jnp.mean(d * d, axis=1, keepdims=True)
    y = d / jnp.sqrt(var + jnp.float32(1e-12))
    o_ref[...] = y * gam_ref[...] + bet_ref[...]


_ln_call = pl.pallas_call(
    _ln_body,
    grid=(N_ROWS // ROWS_BLK,),
    in_specs=[
        pl.BlockSpec((ROWS_BLK, DIM), lambda i: (i, 0)),
        pl.BlockSpec((MAX_LEN, DIM), lambda i: (0, 0)),
        pl.BlockSpec((1, DIM), lambda i: (0, 0)),
        pl.BlockSpec((1, DIM), lambda i: (0, 0)),
    ],
    out_specs=pl.BlockSpec((ROWS_BLK, DIM), lambda i: (i, 0)),
    out_shape=jax.ShapeDtypeStruct((N_ROWS, DIM), jnp.float32),
)


def kernel(embedding, token_table, pos_table, gamma, beta):
    emb2 = embedding.astype(jnp.int32).reshape(NW, CHUNKS_PER_W, CHUNK)
    gathered = _gather_call(emb2, token_table)
    out = _ln_call(gathered, pos_table,
                   gamma.reshape(1, DIM), beta.reshape(1, DIM))
    return out.reshape(B, L, DIM)


# speedup vs baseline: 5.1430x; 1.0379x over previous
import functools

import jax
import jax.numpy as jnp
from jax import lax
from jax.experimental import pallas as pl
from jax.experimental.pallas import tpu as pltpu
from jax.experimental.pallas import tpu_sc as plsc

VOCAB = 100000
DIM = 128
MAX_LEN = 200
B = 1024
L = 200

NC = 2
NS = 16
NW = NC * NS
N_ROWS = B * L
ROWS_PER_W = N_ROWS // NW
CHUNK = 128
CHUNKS_PER_W = ROWS_PER_W // CHUNK
NBUF = 5
ROUNDS = CHUNKS_PER_W // NBUF


def _gather_body(emb_hbm, tok_hbm, out_hbm, idx_v, rows_v, gsem, wsem):
    cid = lax.axis_index("c")
    sid = lax.axis_index("s")
    wid = sid * NC + cid
    base_chunk = wid * CHUNKS_PER_W

    pltpu.sync_copy(emb_hbm.at[wid], idx_v)

    def start_gather(b, c):
        pltpu.async_copy(tok_hbm.at[idx_v.at[c]], rows_v.at[b], gsem.at[b])

    def wait_gather(b, c):
        pltpu.make_async_copy(
            tok_hbm.at[idx_v.at[c]], rows_v.at[b], gsem.at[b]).wait()

    def start_write(b, c):
        row0 = (base_chunk + c) * CHUNK
        pltpu.async_copy(
            rows_v.at[b], out_hbm.at[pl.ds(row0, CHUNK)], wsem.at[b])

    def wait_write(b, c):
        row0 = (base_chunk + c) * CHUNK
        pltpu.make_async_copy(
            rows_v.at[b], out_hbm.at[pl.ds(row0, CHUNK)], wsem.at[b]).wait()

    for b in range(NBUF):
        start_gather(b, b)

    def round_body(r, carry):
        c0 = r * NBUF
        for b in range(NBUF):
            wait_gather(b, c0 + b)
            start_write(b, c0 + b)
        for b in range(NBUF):
            wait_write(b, c0 + b)
            start_gather(b, c0 + NBUF + b)
        return carry

    lax.fori_loop(0, ROUNDS - 1, round_body, 0)

    c0 = (ROUNDS - 1) * NBUF
    for b in range(NBUF):
        wait_gather(b, c0 + b)
        start_write(b, c0 + b)
    for b in range(NBUF):
        wait_write(b, c0 + b)


_gather_call = functools.partial(
    pl.kernel,
    mesh=plsc.VectorSubcoreMesh(core_axis_name="c", subcore_axis_name="s"),
    out_type=jax.ShapeDtypeStruct((N_ROWS, DIM), jnp.float32),
    scratch_types=[
        pltpu.VMEM((CHUNKS_PER_W, CHUNK), jnp.int32),
        pltpu.VMEM((NBUF, CHUNK, DIM), jnp.float32),
        pltpu.SemaphoreType.DMA((NBUF,)),
        pltpu.SemaphoreType.DMA((NBUF,)),
    ],
)(_gather_body)


ROWS_BLK = 12800
REP = ROWS_BLK // MAX_LEN


def _ln_body(g_ref, pos_ref, gam_ref, bet_ref, o_ref):
    x = g_ref[...] + jnp.tile(pos_ref[...], (REP, 1))
    mean = jnp.mean(x, axis=1, keepdims=True)
    d = x - mean
    var = jnp.mean(d * d, axis=1, keepdims=True)
    y = d / jnp.sqrt(var + jnp.float32(1e-12))
    o_ref[...] = y * gam_ref[...] + bet_ref[...]


_ln_call = pl.pallas_call(
    _ln_body,
    grid=(N_ROWS // ROWS_BLK,),
    in_specs=[
        pl.BlockSpec((ROWS_BLK, DIM), lambda i: (i, 0)),
        pl.BlockSpec((MAX_LEN, DIM), lambda i: (0, 0)),
        pl.BlockSpec((1, DIM), lambda i: (0, 0)),
        pl.BlockSpec((1, DIM), lambda i: (0, 0)),
    ],
    out_specs=pl.BlockSpec((ROWS_BLK, DIM), lambda i: (i, 0)),
    out_shape=jax.ShapeDtypeStruct((N_ROWS, DIM), jnp.float32),
)


def kernel(embedding, token_table, pos_table, gamma, beta):
    emb2 = embedding.astype(jnp.int32).reshape(NW, CHUNKS_PER_W, CHUNK)
    gathered = _gather_call(emb2, token_table)
    out = _ln_call(gathered, pos_table,
                   gamma.reshape(1, DIM), beta.reshape(1, DIM))
    return out.reshape(B, L, DIM)


# speedup vs baseline: 5.4111x; 1.0521x over previous
import functools

import jax
import jax.numpy as jnp
from jax import lax
from jax.experimental import pallas as pl
from jax.experimental.pallas import tpu as pltpu
from jax.experimental.pallas import tpu_sc as plsc

VOCAB = 100000
DIM = 128
MAX_LEN = 200
B = 1024
L = 200

NC = 2
NS = 16
NW = NC * NS
N_ROWS = B * L
NSLICE = 2
SLICE_ROWS = N_ROWS // NSLICE
ROWS_PER_W = SLICE_ROWS // NW
CHUNK = 128
CHUNKS_PER_W = ROWS_PER_W // CHUNK
NBUF = 5
ROUNDS = CHUNKS_PER_W // NBUF


def _gather_body(emb_hbm, tok_hbm, out_hbm, idx_v, rows_v, gsem, wsem):
    cid = lax.axis_index("c")
    sid = lax.axis_index("s")
    wid = sid * NC + cid
    base_chunk = wid * CHUNKS_PER_W

    pltpu.sync_copy(emb_hbm.at[wid], idx_v)

    def start_gather(b, c):
        pltpu.async_copy(tok_hbm.at[idx_v.at[c]], rows_v.at[b], gsem.at[b])

    def wait_gather(b, c):
        pltpu.make_async_copy(
            tok_hbm.at[idx_v.at[c]], rows_v.at[b], gsem.at[b]).wait()

    def start_write(b, c):
        row0 = (base_chunk + c) * CHUNK
        pltpu.async_copy(
            rows_v.at[b], out_hbm.at[pl.ds(row0, CHUNK)], wsem.at[b])

    def wait_write(b, c):
        row0 = (base_chunk + c) * CHUNK
        pltpu.make_async_copy(
            rows_v.at[b], out_hbm.at[pl.ds(row0, CHUNK)], wsem.at[b]).wait()

    for b in range(NBUF):
        start_gather(b, b)

    def round_body(r, carry):
        c0 = r * NBUF
        for b in range(NBUF):
            wait_gather(b, c0 + b)
            start_write(b, c0 + b)
        for b in range(NBUF):
            wait_write(b, c0 + b)
            start_gather(b, c0 + NBUF + b)
        return carry

    lax.fori_loop(0, ROUNDS - 1, round_body, 0)

    c0 = (ROUNDS - 1) * NBUF
    for b in range(NBUF):
        wait_gather(b, c0 + b)
        start_write(b, c0 + b)
    for b in range(NBUF):
        wait_write(b, c0 + b)


_gather_call = functools.partial(
    pl.kernel,
    mesh=plsc.VectorSubcoreMesh(core_axis_name="c", subcore_axis_name="s"),
    out_type=jax.ShapeDtypeStruct((SLICE_ROWS, DIM), jnp.float32),
    scratch_types=[
        pltpu.VMEM((CHUNKS_PER_W, CHUNK), jnp.int32),
        pltpu.VMEM((NBUF, CHUNK, DIM), jnp.float32),
        pltpu.SemaphoreType.DMA((NBUF,)),
        pltpu.SemaphoreType.DMA((NBUF,)),
    ],
)(_gather_body)


ROWS_BLK = 12800
REP = ROWS_BLK // MAX_LEN
BLKS_PER_SLICE = SLICE_ROWS // ROWS_BLK


def _ln_math(x, gam, bet):
    mean = jnp.mean(x, axis=1, keepdims=True)
    d = x - mean
    var = jnp.mean(d * d, axis=1, keepdims=True)
    y = d / jnp.sqrt(var + jnp.float32(1e-12))
    return y * gam + bet


def _ln0_body(g_ref, pos_ref, gam_ref, bet_ref, o_ref):
    x = g_ref[...] + jnp.tile(pos_ref[...], (REP, 1))
    o_ref[...] = _ln_math(x, gam_ref[...], bet_ref[...])


def _ln1_body(prev_ref, g_ref, pos_ref, gam_ref, bet_ref, o_ref):
    del prev_ref
    x = g_ref[...] + jnp.tile(pos_ref[...], (REP, 1))
    o_ref[...] = _ln_math(x, gam_ref[...], bet_ref[...])


_ROW_SPEC = pl.BlockSpec((ROWS_BLK, DIM), lambda i: (i, 0))
_POS_SPEC = pl.BlockSpec((MAX_LEN, DIM), lambda i: (0, 0))
_VEC_SPEC = pl.BlockSpec((1, DIM), lambda i: (0, 0))

_ln0_call = pl.pallas_call(
    _ln0_body,
    grid=(BLKS_PER_SLICE,),
    in_specs=[_ROW_SPEC, _POS_SPEC, _VEC_SPEC, _VEC_SPEC],
    out_specs=pl.BlockSpec((ROWS_BLK, DIM), lambda i: (i, 0)),
    out_shape=jax.ShapeDtypeStruct((N_ROWS, DIM), jnp.float32),
)

_ln1_call = pl.pallas_call(
    _ln1_body,
    grid=(BLKS_PER_SLICE,),
    in_specs=[
        pl.BlockSpec(memory_space=pl.ANY),
        _ROW_SPEC, _POS_SPEC, _VEC_SPEC, _VEC_SPEC,
    ],
    out_specs=pl.BlockSpec((ROWS_BLK, DIM), lambda i: (i + BLKS_PER_SLICE, 0)),
    out_shape=jax.ShapeDtypeStruct((N_ROWS, DIM), jnp.float32),
    input_output_aliases={0: 0},
)


def kernel(embedding, token_table, pos_table, gamma, beta):
    emb2 = embedding.astype(jnp.int32).reshape(
        NSLICE, NW, CHUNKS_PER_W, CHUNK)
    g0 = _gather_call(emb2[0], token_table)
    g1 = _gather_call(emb2[1], token_table)
    gam = gamma.reshape(1, DIM)
    bet = beta.reshape(1, DIM)
    out = _ln0_call(g0, pos_table, gam, bet)
    out = _ln1_call(out, g1, pos_table, gam, bet)
    return out.reshape(B, L, DIM)
